# baseline (speedup 1.0000x reference)
import functools

import jax
import jax.numpy as jnp
from jax import lax
from jax.experimental import pallas as pl
from jax.experimental.pallas import tpu as pltpu
from jax.experimental.pallas import tpu_sc as plsc

_FIELDS = 26
_FIELD_SIZE = 100000
_BATCH = 4096
_NC = 2
_NS = 16
_NW = _NC * _NS
_BPW = _BATCH // _NW
_CHUNK = _BPW // 16
_PER_TILE = _BPW * _FIELDS
_F_HALF = _FIELDS // 2

_mesh = plsc.VectorSubcoreMesh(core_axis_name="c", subcore_axis_name="s")


@functools.partial(
    pl.kernel,
    out_type=jax.ShapeDtypeStruct((_BATCH,), jnp.float32),
    mesh=_mesh,
    scratch_types=[
        pltpu.VMEM((_FIELDS, _BPW), jnp.int32),
        pltpu.VMEM((_PER_TILE,), jnp.int32),
        pltpu.VMEM((_PER_TILE,), jnp.float32),
        pltpu.VMEM((_BPW,), jnp.float32),
        pltpu.VMEM((16,), jnp.float32),
        pltpu.SemaphoreType.DMA,
        pltpu.SemaphoreType.DMA,
        pltpu.SemaphoreType.DMA,
    ],
    compiler_params=pltpu.CompilerParams(
        needs_layout_passes=False, skip_device_barrier=True
    ),
)
def _sc_features_linear(xt_hbm, fc_hbm, bias_hbm, out_hbm,
                        xtv, idxv, vals, outv, bias_v, semx, sema, semb):
    wid = lax.axis_index("s") * _NC + lax.axis_index("c")
    base = wid * _BPW

    xcp = pltpu.async_copy(xt_hbm.at[:, pl.ds(base, _BPW)], xtv, semx)
    pltpu.sync_copy(bias_hbm, bias_v.at[pl.ds(0, 1)])
    bvec = jnp.broadcast_to(bias_v[...][0], (16,))
    xcp.wait()

    half = _F_HALF * _BPW
    for f in range(_F_HALF):
        for c in range(_CHUNK):
            idxv[pl.ds(f * _BPW + c * 16, 16)] = (
                xtv[f, pl.ds(c * 16, 16)] + (f * _FIELD_SIZE)
            )
    cpa = pltpu.async_copy(fc_hbm.at[0].at[idxv.at[pl.ds(0, half)]],
                           vals.at[pl.ds(0, half)], sema)
    for f in range(_F_HALF, _FIELDS):
        for c in range(_CHUNK):
            idxv[pl.ds(f * _BPW + c * 16, 16)] = (
                xtv[f, pl.ds(c * 16, 16)] + (f * _FIELD_SIZE)
            )
    cpb = pltpu.async_copy(fc_hbm.at[0].at[idxv.at[pl.ds(half, half)]],
                           vals.at[pl.ds(half, half)], semb)

    cpa.wait()
    for c in range(_CHUNK):
        acc = bvec
        for f in range(_F_HALF):
            acc = acc + vals[pl.ds(f * _BPW + c * 16, 16)]
        outv[pl.ds(c * 16, 16)] = acc
    cpb.wait()
    for c in range(_CHUNK):
        acc = outv[pl.ds(c * 16, 16)]
        for f in range(_F_HALF, _FIELDS):
            acc = acc + vals[pl.ds(f * _BPW + c * 16, 16)]
        outv[pl.ds(c * 16, 16)] = acc

    pltpu.sync_copy(outv, out_hbm.at[pl.ds(base, _BPW)])


def kernel(x, fc, bias):
    xt = x.astype(jnp.int32).T
    fct = fc.astype(jnp.float32).T
    out = _sc_features_linear(xt, fct, bias.astype(jnp.float32))
    return out.reshape(_BATCH, 1)


# speedup vs baseline: 1.0139x; 1.0139x over previous
import functools

import jax
import jax.numpy as jnp
from jax import lax
from jax.experimental import pallas as pl
from jax.experimental.pallas import tpu as pltpu
from jax.experimental.pallas import tpu_sc as plsc

_FIELDS = 26
_FIELD_SIZE = 100000
_BATCH = 4096
_NC = 2
_NS = 16
_NW = _NC * _NS
_BPW = _BATCH // _NW
_CHUNK = _BPW // 16
_PER_TILE = _BPW * _FIELDS
_F_HALF = _FIELDS // 2

_mesh = plsc.VectorSubcoreMesh(core_axis_name="c", subcore_axis_name="s")


@functools.partial(
    pl.kernel,
    out_type=jax.ShapeDtypeStruct((_BATCH,), jnp.float32),
    mesh=_mesh,
    scratch_types=[
        pltpu.VMEM((_FIELDS, _BPW), jnp.int32),
        pltpu.VMEM((_PER_TILE,), jnp.int32),
        pltpu.VMEM((_PER_TILE,), jnp.float32),
        pltpu.VMEM((_BPW,), jnp.float32),
        pltpu.VMEM((16,), jnp.float32),
        pltpu.SemaphoreType.DMA,
        pltpu.SemaphoreType.DMA,
        pltpu.SemaphoreType.DMA,
        pltpu.SemaphoreType.DMA,
        pltpu.SemaphoreType.DMA,
    ],
    compiler_params=pltpu.CompilerParams(needs_layout_passes=False),
)
def _sc_features_linear(xt_hbm, fc_hbm, bias_hbm, out_hbm,
                        xtv, idxv, vals, outv, bias_v,
                        semx1, semx2, sem0, sem1, sem2):
    wid = lax.axis_index("s") * _NC + lax.axis_index("c")
    base = wid * _BPW
    splits = [(0, 8), (8, 17), (17, 26)]
    sems = [sem0, sem1, sem2]

    f0 = splits[0][1]
    xcp1 = pltpu.async_copy(xt_hbm.at[pl.ds(0, f0), pl.ds(base, _BPW)],
                            xtv.at[pl.ds(0, f0)], semx1)
    xcp2 = pltpu.async_copy(xt_hbm.at[pl.ds(f0, _FIELDS - f0), pl.ds(base, _BPW)],
                            xtv.at[pl.ds(f0, _FIELDS - f0)], semx2)
    pltpu.sync_copy(bias_hbm, bias_v.at[pl.ds(0, 1)])
    bvec = jnp.broadcast_to(bias_v[...][0], (16,))

    copies = []
    for s, ((flo, fhi), sem) in enumerate(zip(splits, sems)):
        if s == 0:
            xcp1.wait()
        elif s == 1:
            xcp2.wait()
        for f in range(flo, fhi):
            for c in range(_CHUNK):
                idxv[pl.ds(f * _BPW + c * 16, 16)] = (
                    xtv[f, pl.ds(c * 16, 16)] + (f * _FIELD_SIZE)
                )
        lo, n = flo * _BPW, (fhi - flo) * _BPW
        copies.append(
            pltpu.async_copy(fc_hbm.at[0].at[idxv.at[pl.ds(lo, n)]],
                             vals.at[pl.ds(lo, n)], sem)
        )

    for s, ((flo, fhi), cp) in enumerate(zip(splits, copies)):
        cp.wait()
        for c in range(_CHUNK):
            acc = bvec if s == 0 else outv[pl.ds(c * 16, 16)]
            for f in range(flo, fhi):
                acc = acc + vals[pl.ds(f * _BPW + c * 16, 16)]
            outv[pl.ds(c * 16, 16)] = acc

    pltpu.sync_copy(outv, out_hbm.at[pl.ds(base, _BPW)])


def kernel(x, fc, bias):
    xt = x.astype(jnp.int32).T
    fct = fc.astype(jnp.float32).T
    out = _sc_features_linear(xt, fct, bias.astype(jnp.float32))
    return out.reshape(_BATCH, 1)


# speedup vs baseline: 1.0162x; 1.0022x over previous
import functools

import jax
import jax.numpy as jnp
from jax import lax
from jax.experimental import pallas as pl
from jax.experimental.pallas import tpu as pltpu
from jax.experimental.pallas import tpu_sc as plsc

_FIELDS = 26
_FIELD_SIZE = 100000
_BATCH = 4096
_NC = 2
_NS = 16
_NW = _NC * _NS
_BPW = _BATCH // _NW
_CHUNK = _BPW // 16
_PER_TILE = _BPW * _FIELDS
_F_HALF = _FIELDS // 2

_mesh = plsc.VectorSubcoreMesh(core_axis_name="c", subcore_axis_name="s")


@functools.partial(
    pl.kernel,
    out_type=jax.ShapeDtypeStruct((_BATCH,), jnp.float32),
    mesh=_mesh,
    scratch_types=[
        pltpu.VMEM((_FIELDS, _BPW), jnp.int32),
        pltpu.VMEM((_PER_TILE,), jnp.int32),
        pltpu.VMEM((_PER_TILE,), jnp.float32),
        pltpu.VMEM((_BPW,), jnp.float32),
        pltpu.VMEM((16,), jnp.float32),
        pltpu.SemaphoreType.DMA,
        pltpu.SemaphoreType.DMA,
        pltpu.SemaphoreType.DMA,
        pltpu.SemaphoreType.DMA,
        pltpu.SemaphoreType.DMA,
    ],
    compiler_params=pltpu.CompilerParams(needs_layout_passes=False),
)
def _sc_features_linear(xt_hbm, fc_hbm, bias_hbm, out_hbm,
                        xtv, idxv, vals, outv, bias_v,
                        semx1, semx2, sem0, sem1, sem2):
    wid = lax.axis_index("s") * _NC + lax.axis_index("c")
    base = wid * _BPW
    splits = [(0, 8), (8, 16), (16, 26)]
    sems = [sem0, sem1, sem2]

    f0 = splits[0][1]
    xcp1 = pltpu.async_copy(xt_hbm.at[pl.ds(0, f0), pl.ds(base, _BPW)],
                            xtv.at[pl.ds(0, f0)], semx1)
    xcp2 = pltpu.async_copy(xt_hbm.at[pl.ds(f0, _FIELDS - f0), pl.ds(base, _BPW)],
                            xtv.at[pl.ds(f0, _FIELDS - f0)], semx2)
    pltpu.sync_copy(bias_hbm, bias_v.at[pl.ds(0, 1)])
    bvec = jnp.broadcast_to(bias_v[...][0], (16,))

    copies = []
    for s, ((flo, fhi), sem) in enumerate(zip(splits, sems)):
        if s == 0:
            xcp1.wait()
        elif s == 1:
            xcp2.wait()
        for f in range(flo, fhi):
            for c in range(_CHUNK):
                idxv[pl.ds(f * _BPW + c * 16, 16)] = (
                    xtv[f, pl.ds(c * 16, 16)] + (f * _FIELD_SIZE)
                )
        lo, n = flo * _BPW, (fhi - flo) * _BPW
        copies.append(
            pltpu.async_copy(fc_hbm.at[0].at[idxv.at[pl.ds(lo, n)]],
                             vals.at[pl.ds(lo, n)], sem)
        )

    for s, ((flo, fhi), cp) in enumerate(zip(splits, copies)):
        cp.wait()
        for c in range(_CHUNK):
            acc = bvec if s == 0 else outv[pl.ds(c * 16, 16)]
            for f in range(flo, fhi):
                acc = acc + vals[pl.ds(f * _BPW + c * 16, 16)]
            outv[pl.ds(c * 16, 16)] = acc

    pltpu.sync_copy(outv, out_hbm.at[pl.ds(base, _BPW)])


def kernel(x, fc, bias):
    xt = x.astype(jnp.int32).T
    fct = fc.astype(jnp.float32).T
    out = _sc_features_linear(xt, fct, bias.astype(jnp.float32))
    return out.reshape(_BATCH, 1)
